# Initial kernel scaffold; baseline (speedup 1.0000x reference)
import jax
import jax.numpy as jnp
from jax.experimental import pallas as pl


def kernel(x, rel_emb, loop_rel, w_in, w_out, w_loop, bias, edge_index, edge_type):
    raise NotImplementedError("write your pallas kernel here")


# speedup vs baseline: 6.8415x; 6.8415x over previous
import functools

import jax
import jax.numpy as jnp
from jax import lax
from jax.experimental import pallas as pl
from jax.experimental.pallas import tpu as pltpu
from jax.experimental.pallas import tpu_sc as plsc

N = 10000
E = 320000
HALF = E // 2
D = 128
HD = D // 2

NC = 2
NS = 16
L = 16
W = 8

EDGES_PER_TILE = HALF // NS
CHUNK = 80
NCHUNK = EDGES_PER_TILE // CHUNK
ROWS_PAD = 10240
ROWS_PER_TILE = ROWS_PAD // NS
PIECES = ROWS_PER_TILE // CHUNK

_MESH = plsc.VectorSubcoreMesh(core_axis_name="c", subcore_axis_name="s")


def _sc_deg_kernel(dst2, zeros_deg, ones_deg):
    @functools.partial(
        pl.kernel,
        out_type=jax.ShapeDtypeStruct((NC, ROWS_PAD, D), jnp.float32),
        mesh=_MESH,
        scratch_types=[
            pltpu.VMEM_SHARED((ROWS_PAD, D), jnp.float32),
            pltpu.VMEM((CHUNK,), jnp.int32),
            pltpu.VMEM((CHUNK, D), jnp.float32),
            pltpu.VMEM((CHUNK, D), jnp.float32),
        ],
    )
    def k(dst_h, zdeg_h, ones_h, deg_o, deg_sh, d_v, ones_v, zb):
        c = lax.axis_index("c")
        s = lax.axis_index("s")
        pltpu.sync_copy(zdeg_h, zb)
        pltpu.sync_copy(ones_h, ones_v)
        for piece in range(PIECES):
            r0 = s * ROWS_PER_TILE + piece * CHUNK
            pltpu.sync_copy(zb, deg_sh.at[pl.ds(r0, CHUNK)])
        plsc.subcore_barrier()

        def chunk_body(j, carry):
            off = c * HALF + s * EDGES_PER_TILE + j * CHUNK
            pltpu.sync_copy(dst_h.at[pl.ds(off, CHUNK)], d_v)
            pltpu.sync_copy(ones_v, deg_sh.at[d_v], add=True)
            return carry

        lax.fori_loop(0, NCHUNK, chunk_body, 0)
        plsc.subcore_barrier()
        for piece in range(PIECES):
            r0 = s * ROWS_PER_TILE + piece * CHUNK
            pltpu.sync_copy(deg_sh.at[pl.ds(r0, CHUNK)], zb)
            pltpu.sync_copy(zb, deg_o.at[c, pl.ds(r0, CHUNK)])

    return k(dst2, zeros_deg, ones_deg)


def _sc_edge_kernel(src2, dst2, typ2, x, rel_emb, zeros_row):
    @functools.partial(
        pl.kernel,
        out_type=jax.ShapeDtypeStruct((NC, ROWS_PAD, D), jnp.float32),
        mesh=_MESH,
        scratch_types=[
            pltpu.VMEM_SHARED((ROWS_PAD, D), jnp.float32),
            pltpu.VMEM((CHUNK,), jnp.int32),
            pltpu.VMEM((CHUNK,), jnp.int32),
            pltpu.VMEM((CHUNK,), jnp.int32),
            pltpu.VMEM((CHUNK, D), jnp.float32),
            pltpu.VMEM((CHUNK, D), jnp.float32),
            pltpu.SemaphoreType.DMA,
            pltpu.SemaphoreType.DMA,
        ],
    )
    def k(src_h, dst_h, typ_h, x_h, rel_h, zrow_h,
          agg_o, agg_sh, s_v, d_v, t_v, xb, rb, sem_x, sem_r):
        c = lax.axis_index("c")
        s = lax.axis_index("s")

        pltpu.sync_copy(zrow_h, xb)
        for piece in range(PIECES):
            r0 = s * ROWS_PER_TILE + piece * CHUNK
            pltpu.sync_copy(xb, agg_sh.at[pl.ds(r0, CHUNK)])
        plsc.subcore_barrier()

        def chunk_body(j, carry):
            off = c * HALF + s * EDGES_PER_TILE + j * CHUNK
            pltpu.sync_copy(src_h.at[pl.ds(off, CHUNK)], s_v)
            pltpu.sync_copy(dst_h.at[pl.ds(off, CHUNK)], d_v)
            pltpu.sync_copy(typ_h.at[pl.ds(off, CHUNK)], t_v)
            cp_x = pltpu.async_copy(x_h.at[s_v], xb, sem_x)
            cp_r = pltpu.async_copy(rel_h.at[t_v], rb, sem_r)
            cp_x.wait()
            cp_r.wait()

            def edge_body(e, carry2):
                for kk in range(HD // L):
                    re_sl = pl.ds(kk * L, L)
                    im_sl = pl.ds(HD + kk * L, L)
                    hre = xb[e, re_sl]
                    him = xb[e, im_sl]
                    rre = rb[e, re_sl]
                    rim = rb[e, im_sl]
                    xb[e, re_sl] = hre * rre - him * rim
                    xb[e, im_sl] = hre * rim + him * rre
                return carry2

            lax.fori_loop(0, CHUNK, edge_body, 0)
            pltpu.sync_copy(xb, agg_sh.at[d_v], add=True)
            return carry

        lax.fori_loop(0, NCHUNK, chunk_body, 0)
        plsc.subcore_barrier()

        for piece in range(PIECES):
            r0 = s * ROWS_PER_TILE + piece * CHUNK
            pltpu.sync_copy(agg_sh.at[pl.ds(r0, CHUNK)], xb)
            pltpu.sync_copy(xb, agg_o.at[c, pl.ds(r0, CHUNK)])

    return k(src2, dst2, typ2, x, rel_emb, zeros_row)


def _tc_combine_kernel(agg_in, agg_out, deg_in, deg_out, x, loop_rel2,
                       w_in, w_out, w_loop, bias2):
    BLK = 1000

    def body(ai_ref, ao_ref, di_ref, do_ref, x_ref, lr_ref,
             wi_ref, wo_ref, wl_ref, b_ref, o_ref):
        ni = 1.0 / jnp.maximum(di_ref[:, 0:1], 1.0)
        no = 1.0 / jnp.maximum(do_ref[:, 0:1], 1.0)
        a = ai_ref[...] * ni
        b = ao_ref[...] * no
        xr = x_ref[...]
        lr = lr_ref[...]
        hre, him = xr[:, :HD], xr[:, HD:]
        rre, rim = lr[:, :HD], lr[:, HD:]
        rot = jnp.concatenate([hre * rre - him * rim,
                               hre * rim + him * rre], axis=1)
        acc = (jnp.dot(a, wi_ref[...], preferred_element_type=jnp.float32)
               + jnp.dot(b, wo_ref[...], preferred_element_type=jnp.float32)
               + jnp.dot(rot, wl_ref[...], preferred_element_type=jnp.float32))
        o_ref[...] = jnp.tanh(acc * (1.0 / 3.0) + b_ref[...])

    grid = (N // BLK,)
    row_spec = pl.BlockSpec((BLK, D), lambda i: (i, 0))
    deg_spec = pl.BlockSpec((BLK, D), lambda i: (i, 0))
    full_spec = pl.BlockSpec((D, D), lambda i: (0, 0))
    vec_spec = pl.BlockSpec((1, D), lambda i: (0, 0))
    return pl.pallas_call(
        body,
        grid=grid,
        in_specs=[row_spec, row_spec, deg_spec, deg_spec, row_spec, vec_spec,
                  full_spec, full_spec, full_spec, vec_spec],
        out_specs=row_spec,
        out_shape=jax.ShapeDtypeStruct((N, D), jnp.float32),
    )(agg_in, agg_out, deg_in, deg_out, x, loop_rel2, w_in, w_out, w_loop,
      bias2)


def kernel(x, rel_emb, loop_rel, w_in, w_out, w_loop, bias, edge_index, edge_type):
    src2 = edge_index[0]
    dst2 = edge_index[1]
    typ2 = edge_type
    zeros_row = jnp.zeros((CHUNK, D), jnp.float32)
    zeros_deg = jnp.zeros((CHUNK, D), jnp.float32)
    ones_deg = jnp.ones((CHUNK, D), jnp.float32)

    deg = _sc_deg_kernel(dst2, zeros_deg, ones_deg)
    agg = _sc_edge_kernel(src2, dst2, typ2, x, rel_emb, zeros_row)

    out = _tc_combine_kernel(agg[0, :N], agg[1, :N], deg[0, :N], deg[1, :N],
                             x, loop_rel.reshape(1, D),
                             w_in, w_out, w_loop, bias.reshape(1, D))
    return out


# speedup vs baseline: 6.8678x; 1.0038x over previous
import functools

import jax
import jax.numpy as jnp
from jax import lax
from jax.experimental import pallas as pl
from jax.experimental.pallas import tpu as pltpu
from jax.experimental.pallas import tpu_sc as plsc

N = 10000
E = 320000
HALF = E // 2
D = 128
HD = D // 2

NC = 2
NS = 16
L = 16
W = 8

EDGES_PER_TILE = HALF // NS
CHUNK = 80
NCHUNK = EDGES_PER_TILE // CHUNK
CE = 80
NCHUNK_E = EDGES_PER_TILE // CE
ROWS_PAD = 10240
ROWS_PER_TILE = ROWS_PAD // NS
PIECES = ROWS_PER_TILE // CHUNK

_MESH = plsc.VectorSubcoreMesh(core_axis_name="c", subcore_axis_name="s")


def _sc_deg_kernel(dst2, zeros_deg, ones_deg):
    @functools.partial(
        pl.kernel,
        out_type=jax.ShapeDtypeStruct((NC, ROWS_PAD, D), jnp.float32),
        mesh=_MESH,
        scratch_types=[
            pltpu.VMEM_SHARED((ROWS_PAD, D), jnp.float32),
            pltpu.VMEM((CHUNK,), jnp.int32),
            pltpu.VMEM((CHUNK, D), jnp.float32),
            pltpu.VMEM((CHUNK, D), jnp.float32),
        ],
    )
    def k(dst_h, zdeg_h, ones_h, deg_o, deg_sh, d_v, ones_v, zb):
        c = lax.axis_index("c")
        s = lax.axis_index("s")
        pltpu.sync_copy(zdeg_h, zb)
        pltpu.sync_copy(ones_h, ones_v)
        for piece in range(PIECES):
            r0 = s * ROWS_PER_TILE + piece * CHUNK
            pltpu.sync_copy(zb, deg_sh.at[pl.ds(r0, CHUNK)])
        plsc.subcore_barrier()

        def chunk_body(j, carry):
            off = c * HALF + s * EDGES_PER_TILE + j * CHUNK
            pltpu.sync_copy(dst_h.at[pl.ds(off, CHUNK)], d_v)
            pltpu.sync_copy(ones_v, deg_sh.at[d_v], add=True)
            return carry

        lax.fori_loop(0, NCHUNK, chunk_body, 0)
        plsc.subcore_barrier()
        for piece in range(PIECES):
            r0 = s * ROWS_PER_TILE + piece * CHUNK
            pltpu.sync_copy(deg_sh.at[pl.ds(r0, CHUNK)], zb)
            pltpu.sync_copy(zb, deg_o.at[c, pl.ds(r0, CHUNK)])

    return k(dst2, zeros_deg, ones_deg)


def _sc_edge_kernel(src2, dst2, typ2, x, rel_emb):
    @functools.partial(
        pl.kernel,
        out_type=jax.ShapeDtypeStruct((NC, ROWS_PAD, D), jnp.float32),
        mesh=_MESH,
        scratch_types=[
            pltpu.VMEM_SHARED((ROWS_PAD, D), jnp.float32),
            pltpu.VMEM((CE,), jnp.int32),
            pltpu.VMEM((CE,), jnp.int32),
            pltpu.VMEM((CE,), jnp.int32),
            pltpu.VMEM((CE, D), jnp.float32),
            pltpu.VMEM((CE, D), jnp.float32),
            pltpu.SemaphoreType.DMA,
            pltpu.SemaphoreType.DMA,
        ],
    )
    def k(src_h, dst_h, typ_h, x_h, rel_h,
          agg_o, agg_sh, s_v, d_v, t_v, xb, rb, gx, gr):
        c = lax.axis_index("c")
        s = lax.axis_index("s")

        zv = jnp.zeros((L,), jnp.float32)

        def zero_row(e, carry0):
            for kk in range(D // L):
                xb[e, pl.ds(kk * L, L)] = zv
            return carry0

        lax.fori_loop(0, CE, zero_row, 0)
        for piece in range(ROWS_PER_TILE // CE):
            r0 = s * ROWS_PER_TILE + piece * CE
            pltpu.sync_copy(xb, agg_sh.at[pl.ds(r0, CE)])
        plsc.subcore_barrier()

        def rotate_buf(e, carry2):
            for kk in range(HD // L):
                re_sl = pl.ds(kk * L, L)
                im_sl = pl.ds(HD + kk * L, L)
                hre = xb[e, re_sl]
                him = xb[e, im_sl]
                rre = rb[e, re_sl]
                rim = rb[e, im_sl]
                xb[e, re_sl] = hre * rre - him * rim
                xb[e, im_sl] = hre * rim + him * rre
            return carry2

        def chunk_body(j, carry):
            off = c * HALF + s * EDGES_PER_TILE + j * CE
            pltpu.sync_copy(src_h.at[pl.ds(off, CE)], s_v)
            pltpu.sync_copy(typ_h.at[pl.ds(off, CE)], t_v)
            pltpu.sync_copy(dst_h.at[pl.ds(off, CE)], d_v)
            pltpu.async_copy(x_h.at[s_v], xb, gx)
            pltpu.async_copy(rel_h.at[t_v], rb, gr)
            pltpu.make_async_copy(x_h.at[s_v], xb, gx).wait()
            pltpu.make_async_copy(rel_h.at[t_v], rb, gr).wait()
            lax.fori_loop(0, CE, rotate_buf, 0)
            pltpu.sync_copy(xb, agg_sh.at[d_v], add=True)
            return carry

        lax.fori_loop(0, NCHUNK_E, chunk_body, 0)
        plsc.subcore_barrier()

        for piece in range(ROWS_PER_TILE // CE):
            r0 = s * ROWS_PER_TILE + piece * CE
            pltpu.sync_copy(agg_sh.at[pl.ds(r0, CE)], xb)
            pltpu.sync_copy(xb, agg_o.at[c, pl.ds(r0, CE)])

    return k(src2, dst2, typ2, x, rel_emb)


def _tc_combine_kernel(agg_in, agg_out, deg_in, deg_out, x, loop_rel2,
                       w_in, w_out, w_loop, bias2):
    BLK = 1000

    def body(ai_ref, ao_ref, di_ref, do_ref, x_ref, lr_ref,
             wi_ref, wo_ref, wl_ref, b_ref, o_ref):
        ni = 1.0 / jnp.maximum(di_ref[:, 0:1], 1.0)
        no = 1.0 / jnp.maximum(do_ref[:, 0:1], 1.0)
        a = ai_ref[...] * ni
        b = ao_ref[...] * no
        xr = x_ref[...]
        lr = lr_ref[...]
        hre, him = xr[:, :HD], xr[:, HD:]
        rre, rim = lr[:, :HD], lr[:, HD:]
        rot = jnp.concatenate([hre * rre - him * rim,
                               hre * rim + him * rre], axis=1)
        acc = (jnp.dot(a, wi_ref[...], preferred_element_type=jnp.float32)
               + jnp.dot(b, wo_ref[...], preferred_element_type=jnp.float32)
               + jnp.dot(rot, wl_ref[...], preferred_element_type=jnp.float32))
        o_ref[...] = jnp.tanh(acc * (1.0 / 3.0) + b_ref[...])

    grid = (N // BLK,)
    row_spec = pl.BlockSpec((BLK, D), lambda i: (i, 0))
    deg_spec = pl.BlockSpec((BLK, D), lambda i: (i, 0))
    full_spec = pl.BlockSpec((D, D), lambda i: (0, 0))
    vec_spec = pl.BlockSpec((1, D), lambda i: (0, 0))
    return pl.pallas_call(
        body,
        grid=grid,
        in_specs=[row_spec, row_spec, deg_spec, deg_spec, row_spec, vec_spec,
                  full_spec, full_spec, full_spec, vec_spec],
        out_specs=row_spec,
        out_shape=jax.ShapeDtypeStruct((N, D), jnp.float32),
    )(agg_in, agg_out, deg_in, deg_out, x, loop_rel2, w_in, w_out, w_loop,
      bias2)


def kernel(x, rel_emb, loop_rel, w_in, w_out, w_loop, bias, edge_index, edge_type):
    src2 = edge_index[0]
    dst2 = edge_index[1]
    typ2 = edge_type
    zeros_deg = jnp.zeros((CHUNK, D), jnp.float32)
    ones_deg = jnp.ones((CHUNK, D), jnp.float32)

    deg = _sc_deg_kernel(dst2, zeros_deg, ones_deg)
    agg = _sc_edge_kernel(src2, dst2, typ2, x, rel_emb)

    out = _tc_combine_kernel(agg[0, :N], agg[1, :N], deg[0, :N], deg[1, :N],
                             x, loop_rel.reshape(1, D),
                             w_in, w_out, w_loop, bias.reshape(1, D))
    return out


# speedup vs baseline: 6.9128x; 1.0066x over previous
import functools

import jax
import jax.numpy as jnp
from jax import lax
from jax.experimental import pallas as pl
from jax.experimental.pallas import tpu as pltpu
from jax.experimental.pallas import tpu_sc as plsc

N = 10000
E = 320000
HALF = E // 2
D = 128
HD = D // 2

NC = 2
NS = 16
L = 16
W = 16

EDGES_PER_TILE = HALF // NS
CHUNK = 80
NCHUNK = EDGES_PER_TILE // CHUNK
CE = 80
NCHUNK_E = EDGES_PER_TILE // CE
ROWS_PAD = 10240
ROWS_PER_TILE = ROWS_PAD // NS
PIECES = ROWS_PER_TILE // CHUNK

_MESH = plsc.VectorSubcoreMesh(core_axis_name="c", subcore_axis_name="s")


def _sc_deg_kernel(dst2):
    @functools.partial(
        pl.kernel,
        out_type=jax.ShapeDtypeStruct((NC, ROWS_PAD, D), jnp.float32),
        mesh=_MESH,
        scratch_types=[
            pltpu.VMEM_SHARED((ROWS_PAD, D), jnp.float32),
            pltpu.VMEM((CHUNK,), jnp.int32),
            pltpu.VMEM((CHUNK, D), jnp.float32),
            pltpu.VMEM((CHUNK, D), jnp.float32),
        ],
    )
    def k(dst_h, deg_o, deg_sh, d_v, ones_v, zb):
        c = lax.axis_index("c")
        s = lax.axis_index("s")
        zv = jnp.zeros((L,), jnp.float32)
        ov = jnp.ones((L,), jnp.float32)

        def fill_row(e, carry0):
            for kk in range(D // L):
                zb[e, pl.ds(kk * L, L)] = zv
                ones_v[e, pl.ds(kk * L, L)] = ov
            return carry0

        lax.fori_loop(0, CHUNK, fill_row, 0)
        for piece in range(PIECES):
            r0 = s * ROWS_PER_TILE + piece * CHUNK
            pltpu.sync_copy(zb, deg_sh.at[pl.ds(r0, CHUNK)])
        plsc.subcore_barrier()

        def chunk_body(j, carry):
            off = c * HALF + s * EDGES_PER_TILE + j * CHUNK
            pltpu.sync_copy(dst_h.at[pl.ds(off, CHUNK)], d_v)
            pltpu.sync_copy(ones_v, deg_sh.at[d_v], add=True)
            return carry

        lax.fori_loop(0, NCHUNK, chunk_body, 0)
        plsc.subcore_barrier()
        for piece in range(PIECES):
            r0 = s * ROWS_PER_TILE + piece * CHUNK
            pltpu.sync_copy(deg_sh.at[pl.ds(r0, CHUNK)], zb)
            pltpu.sync_copy(zb, deg_o.at[c, pl.ds(r0, CHUNK)])

    return k(dst2)


def _sc_edge_kernel(src2, dst2, typ2, x, rel_emb):
    @functools.partial(
        pl.kernel,
        out_type=jax.ShapeDtypeStruct((NC, ROWS_PAD, D), jnp.float32),
        mesh=_MESH,
        scratch_types=[
            pltpu.VMEM_SHARED((ROWS_PAD, D), jnp.float32),
            pltpu.VMEM((CE,), jnp.int32),
            pltpu.VMEM((CE,), jnp.int32),
            pltpu.VMEM((CE,), jnp.int32),
            pltpu.VMEM((CE, D), jnp.float32),
            pltpu.VMEM((CE, D), jnp.float32),
            pltpu.SemaphoreType.DMA,
            pltpu.SemaphoreType.DMA,
        ],
    )
    def k(src_h, dst_h, typ_h, x_h, rel_h,
          agg_o, agg_sh, s_v, d_v, t_v, xb, rb, gx, gr):
        c = lax.axis_index("c")
        s = lax.axis_index("s")

        zv = jnp.zeros((L,), jnp.float32)

        def zero_row(e, carry0):
            for kk in range(D // L):
                xb[e, pl.ds(kk * L, L)] = zv
            return carry0

        lax.fori_loop(0, CE, zero_row, 0)
        for piece in range(ROWS_PER_TILE // CE):
            r0 = s * ROWS_PER_TILE + piece * CE
            pltpu.sync_copy(xb, agg_sh.at[pl.ds(r0, CE)])
        plsc.subcore_barrier()

        def rotate_buf(e, carry2):
            for kk in range(HD // L):
                re_sl = pl.ds(kk * L, L)
                im_sl = pl.ds(HD + kk * L, L)
                hre = xb[e, re_sl]
                him = xb[e, im_sl]
                rre = rb[e, re_sl]
                rim = rb[e, im_sl]
                xb[e, re_sl] = hre * rre - him * rim
                xb[e, im_sl] = hre * rim + him * rre
            return carry2

        def chunk_body(j, carry):
            off = c * HALF + s * EDGES_PER_TILE + j * CE
            pltpu.sync_copy(src_h.at[pl.ds(off, CE)], s_v)
            pltpu.sync_copy(typ_h.at[pl.ds(off, CE)], t_v)
            pltpu.sync_copy(dst_h.at[pl.ds(off, CE)], d_v)
            pltpu.async_copy(x_h.at[s_v], xb, gx)
            pltpu.async_copy(rel_h.at[t_v], rb, gr)
            pltpu.make_async_copy(x_h.at[s_v], xb, gx).wait()
            pltpu.make_async_copy(rel_h.at[t_v], rb, gr).wait()
            lax.fori_loop(0, CE, rotate_buf, 0)
            pltpu.sync_copy(xb, agg_sh.at[d_v], add=True)
            return carry

        lax.fori_loop(0, NCHUNK_E, chunk_body, 0)
        plsc.subcore_barrier()

        for piece in range(ROWS_PER_TILE // CE):
            r0 = s * ROWS_PER_TILE + piece * CE
            pltpu.sync_copy(agg_sh.at[pl.ds(r0, CE)], xb)
            pltpu.sync_copy(xb, agg_o.at[c, pl.ds(r0, CE)])

    return k(src2, dst2, typ2, x, rel_emb)


def _tc_combine_kernel(agg_in, agg_out, deg_in, deg_out, x, loop_rel2,
                       w_in, w_out, w_loop, bias2):
    BLK = 1000

    def body(ai_ref, ao_ref, di_ref, do_ref, x_ref, lr_ref,
             wi_ref, wo_ref, wl_ref, b_ref, o_ref):
        ni = 1.0 / jnp.maximum(di_ref[:, 0:1], 1.0)
        no = 1.0 / jnp.maximum(do_ref[:, 0:1], 1.0)
        a = ai_ref[...] * ni
        b = ao_ref[...] * no
        xr = x_ref[...]
        lr = lr_ref[...]
        hre, him = xr[:, :HD], xr[:, HD:]
        rre, rim = lr[:, :HD], lr[:, HD:]
        rot = jnp.concatenate([hre * rre - him * rim,
                               hre * rim + him * rre], axis=1)
        acc = (jnp.dot(a, wi_ref[...], preferred_element_type=jnp.float32)
               + jnp.dot(b, wo_ref[...], preferred_element_type=jnp.float32)
               + jnp.dot(rot, wl_ref[...], preferred_element_type=jnp.float32))
        o_ref[...] = jnp.tanh(acc * (1.0 / 3.0) + b_ref[...])

    grid = (N // BLK,)
    row_spec = pl.BlockSpec((BLK, D), lambda i: (i, 0))
    deg_spec = pl.BlockSpec((BLK, D), lambda i: (i, 0))
    full_spec = pl.BlockSpec((D, D), lambda i: (0, 0))
    vec_spec = pl.BlockSpec((1, D), lambda i: (0, 0))
    return pl.pallas_call(
        body,
        grid=grid,
        in_specs=[row_spec, row_spec, deg_spec, deg_spec, row_spec, vec_spec,
                  full_spec, full_spec, full_spec, vec_spec],
        out_specs=row_spec,
        out_shape=jax.ShapeDtypeStruct((N, D), jnp.float32),
    )(agg_in, agg_out, deg_in, deg_out, x, loop_rel2, w_in, w_out, w_loop,
      bias2)


def kernel(x, rel_emb, loop_rel, w_in, w_out, w_loop, bias, edge_index, edge_type):
    src2 = edge_index[0]
    dst2 = edge_index[1]
    typ2 = edge_type

    deg = _sc_deg_kernel(dst2)
    agg = _sc_edge_kernel(src2, dst2, typ2, x, rel_emb)

    out = _tc_combine_kernel(agg[0, :N], agg[1, :N], deg[0, :N], deg[1, :N],
                             x, loop_rel.reshape(1, D),
                             w_in, w_out, w_loop, bias.reshape(1, D))
    return out
